# Initial kernel scaffold; baseline (speedup 1.0000x reference)
import jax
import jax.numpy as jnp
from jax.experimental import pallas as pl


def kernel(hidden_states, router_weight, expert_gate_w, expert_up_w, expert_down_w, shared_gate_w, shared_up_w, shared_down_w, shared_gate_lin_w):
    raise NotImplementedError("write your pallas kernel here")


# speedup vs baseline: 1.3892x; 1.3892x over previous
import functools

import jax
import jax.numpy as jnp
from jax import lax
from jax.experimental import pallas as pl
from jax.experimental.pallas import tpu as pltpu
from jax.experimental.pallas import tpu_sc as plsc

E = 8
TOPK = 2
D = 2048
FF = 512
FFS = 512
T = 2048
TILE_M = 128
NT = 40
ROWS = NT * TILE_M
NW = 32
TPW = T // NW
CHT = 16
NCH = TPW // CHT


def _routing_body(x_ref, rw_ref, dst_ref, wts_ref, meta_ref, ws_ref, contrib_ref, cum_ref):
    x = x_ref[...]
    rw = rw_ref[...]
    logits = lax.dot_general(x, rw, (((1,), (1,)), ((), ())),
                             preferred_element_type=jnp.float32)
    m = jnp.max(logits, axis=1, keepdims=True)
    ex = jnp.exp(logits - m)
    probs = ex / jnp.sum(ex, axis=1, keepdims=True)

    iota8 = lax.broadcasted_iota(jnp.int32, (T, E), 1)
    v1 = jnp.max(probs, axis=1, keepdims=True)
    i1 = jnp.min(jnp.where(probs == v1, iota8, E), axis=1, keepdims=True)
    p2 = jnp.where(iota8 == i1, -1.0, probs)
    v2 = jnp.max(p2, axis=1, keepdims=True)
    i2 = jnp.min(jnp.where(p2 == v2, iota8, E), axis=1, keepdims=True)
    denom = v1 + v2
    w1 = v1 / denom
    w2 = v2 / denom

    oh1 = (iota8 == i1).astype(jnp.float32)
    oh2 = (iota8 == i2).astype(jnp.float32)
    contrib_ref[...] = oh1 + oh2

    CH = 256
    r = lax.broadcasted_iota(jnp.int32, (CH, CH), 0)
    c = lax.broadcasted_iota(jnp.int32, (CH, CH), 1)
    tri = (r > c).astype(jnp.float32)

    def step(ch, carry):
        blk = contrib_ref[pl.ds(ch * CH, CH), :]
        cum_blk = lax.dot_general(tri, blk, (((1,), (0,)), ((), ())),
                                  preferred_element_type=jnp.float32)
        cum_ref[pl.ds(ch * CH, CH), :] = cum_blk + carry
        return carry + jnp.sum(blk, axis=0, keepdims=True)

    counts = lax.fori_loop(0, T // CH, step, jnp.zeros((1, E), jnp.float32))

    ci = counts.astype(jnp.int32)
    ntiles = (ci + (TILE_M - 1)) // TILE_M
    e_r = lax.broadcasted_iota(jnp.int32, (E, E), 0)
    e_c = lax.broadcasted_iota(jnp.int32, (E, E), 1)
    mlt = (e_r < e_c).astype(jnp.float32)
    tile_off = lax.dot_general(ntiles.astype(jnp.float32), mlt,
                               (((1,), (0,)), ((), ())),
                               preferred_element_type=jnp.float32)
    row_off = tile_off * float(TILE_M)

    cum = cum_ref[...]
    sel1 = (iota8 == i1).astype(jnp.float32)
    sel2 = (iota8 == i2).astype(jnp.float32)
    dst0 = jnp.sum((cum + row_off) * sel1, axis=1, keepdims=True)
    dst1 = jnp.sum((cum + row_off) * sel2, axis=1, keepdims=True)

    colsel0 = (iota8 == 0).astype(jnp.float32)
    colsel1 = (iota8 == 1).astype(jnp.float32)
    dst_ref[...] = (dst0 * colsel0 + dst1 * colsel1).astype(jnp.int32)
    wts_ref[...] = w1 * colsel0 + w2 * colsel1

    WCH = 512
    piota = lax.broadcasted_iota(jnp.int32, (T, WCH), 1).astype(jnp.float32)
    for c in range(ROWS // WCH):
        pc = piota + float(c * WCH)
        contrib_w = (jnp.where(dst0 == pc, w1, 0.0)
                     + jnp.where(dst1 == pc, w2, 0.0))
        ws_ref[pl.ds(c, 1), :] = jnp.sum(contrib_w, axis=0, keepdims=True)

    tile_end = (tile_off + ntiles.astype(jnp.float32))
    i8 = (e_r == e_c).astype(jnp.float32)
    ends_col = lax.dot_general(i8, tile_end, (((1,), (1,)), ((), ())),
                               preferred_element_type=jnp.float32)
    ends_b = jnp.broadcast_to(ends_col, (E, 128))
    jot = lax.broadcasted_iota(jnp.int32, (E, 128), 1).astype(jnp.float32)
    eid = jnp.sum((ends_b <= jot).astype(jnp.float32), axis=0, keepdims=True)
    eid = jnp.minimum(eid, float(E - 1))
    n_active = jnp.sum(ntiles.astype(jnp.float32), axis=1, keepdims=True)
    lane = lax.broadcasted_iota(jnp.int32, (1, 128), 1)
    meta_ref[...] = jnp.where(lane == 120, n_active, eid).astype(jnp.int32)


def _routing(x, router_weight):
    return pl.pallas_call(
        _routing_body,
        out_shape=(
            jax.ShapeDtypeStruct((T, E), jnp.int32),
            jax.ShapeDtypeStruct((T, E), jnp.float32),
            jax.ShapeDtypeStruct((1, 128), jnp.int32),
            jax.ShapeDtypeStruct((ROWS // 512, 512), jnp.float32),
        ),
        scratch_shapes=[
            pltpu.VMEM((T, E), jnp.float32),
            pltpu.VMEM((T, E), jnp.float32),
        ],
    )(x, router_weight)


def _shared_body(x_ref, gw_ref, uw_ref, dw_ref, gl_ref, out_ref):
    x = x_ref[...]
    g = lax.dot_general(x, gw_ref[...], (((1,), (1,)), ((), ())),
                        preferred_element_type=jnp.float32)
    u = lax.dot_general(x, uw_ref[...], (((1,), (1,)), ((), ())),
                        preferred_element_type=jnp.float32)
    h = g * jax.nn.sigmoid(g) * u
    y = lax.dot_general(h, dw_ref[...], (((1,), (1,)), ((), ())),
                        preferred_element_type=jnp.float32)
    gate = jax.nn.sigmoid(lax.dot_general(x, gl_ref[...], (((1,), (1,)), ((), ())),
                                          preferred_element_type=jnp.float32))
    out_ref[...] = gate * y


def _shared(x, gw, uw, dw, gl):
    BT = 256
    return pl.pallas_call(
        _shared_body,
        grid=(T // BT,),
        in_specs=[
            pl.BlockSpec((BT, D), lambda i: (i, 0)),
            pl.BlockSpec((FFS, D), lambda i: (0, 0)),
            pl.BlockSpec((FFS, D), lambda i: (0, 0)),
            pl.BlockSpec((D, FFS), lambda i: (0, 0)),
            pl.BlockSpec((1, D), lambda i: (0, 0)),
        ],
        out_specs=pl.BlockSpec((BT, D), lambda i: (i, 0)),
        out_shape=jax.ShapeDtypeStruct((T, D), jnp.float32),
    )(x, gw, uw, dw, gl)


def _dispatch(x, i0, i1):
    mesh = plsc.VectorSubcoreMesh(core_axis_name="c", subcore_axis_name="s", num_cores=2, num_subcores=16)

    @functools.partial(
        pl.kernel,
        mesh=mesh,
        out_type=jax.ShapeDtypeStruct((ROWS, D), jnp.float32),
        scratch_types=[
            pltpu.VMEM((CHT, D), jnp.float32),
            pltpu.VMEM((CHT,), jnp.int32),
            pltpu.VMEM((CHT,), jnp.int32),
            pltpu.SemaphoreType.DMA,
        ],
    )
    def body(x_hbm, i0_hbm, i1_hbm, xs_hbm, rows_v, idx0_v, idx1_v, sem):
        nc = 2
        wid = lax.axis_index("s") * nc + lax.axis_index("c")

        def chunk(c, _):
            base = wid * TPW + c * CHT
            pltpu.sync_copy(x_hbm.at[pl.ds(base, CHT)], rows_v)
            pltpu.sync_copy(i0_hbm.at[wid, c], idx0_v)
            pltpu.sync_copy(i1_hbm.at[wid, c], idx1_v)
            cp0 = pltpu.async_copy(rows_v, xs_hbm.at[idx0_v], sem)
            cp1 = pltpu.async_copy(rows_v, xs_hbm.at[idx1_v], sem)
            cp0.wait()
            cp1.wait()
            return 0

        lax.fori_loop(0, NCH, chunk, 0)

    return body(x, i0, i1)


def _grouped_body(sp_ref, xs_ref, gw_ref, uw_ref, dw_ref, ws_ref, out_ref):
    i = pl.program_id(0)
    n_active = sp_ref[120]

    @pl.when(i < n_active)
    def _():
        xb = xs_ref[...]
        gwe = gw_ref[0]
        uwe = uw_ref[0]
        dwe = dw_ref[0]
        g = lax.dot_general(xb, gwe, (((1,), (1,)), ((), ())),
                            preferred_element_type=jnp.float32)
        u = lax.dot_general(xb, uwe, (((1,), (1,)), ((), ())),
                            preferred_element_type=jnp.float32)
        h = g * jax.nn.sigmoid(g) * u
        h = h * ws_ref[...]
        out_ref[...] = lax.dot_general(h, dwe, (((1,), (1,)), ((), ())),
                                       preferred_element_type=jnp.float32)


def _grouped(meta128, xs, gw, uw, dw, ws2d):
    grid_spec = pltpu.PrefetchScalarGridSpec(
        num_scalar_prefetch=1,
        grid=(NT,),
        in_specs=[
            pl.BlockSpec((TILE_M, D), lambda i, sp: (i, 0)),
            pl.BlockSpec((1, FF, D), lambda i, sp: (sp[i], 0, 0)),
            pl.BlockSpec((1, FF, D), lambda i, sp: (sp[i], 0, 0)),
            pl.BlockSpec((1, D, FF), lambda i, sp: (sp[i], 0, 0)),
            pl.BlockSpec((TILE_M, 1), lambda i, sp: (i, 0)),
        ],
        out_specs=pl.BlockSpec((TILE_M, D), lambda i, sp: (i, 0)),
    )
    return pl.pallas_call(
        _grouped_body,
        grid_spec=grid_spec,
        out_shape=jax.ShapeDtypeStruct((ROWS, D), jnp.float32),
    )(meta128, xs, gw, uw, dw, ws2d)


def _combine(ys, shared_pre, i0, i1):
    mesh = plsc.VectorSubcoreMesh(core_axis_name="c", subcore_axis_name="s", num_cores=2, num_subcores=16)

    @functools.partial(
        pl.kernel,
        mesh=mesh,
        out_type=jax.ShapeDtypeStruct((T, D), jnp.float32),
        scratch_types=[
            pltpu.VMEM((CHT, D), jnp.float32),
            pltpu.VMEM((CHT, D), jnp.float32),
            pltpu.VMEM((CHT, D), jnp.float32),
            pltpu.VMEM((CHT,), jnp.int32),
            pltpu.VMEM((CHT,), jnp.int32),
            pltpu.SemaphoreType.DMA,
        ],
    )
    def body(ys_hbm, sp_hbm, i0_hbm, i1_hbm, out_hbm,
             acc_v, r0_v, r1_v, idx0_v, idx1_v, sem):
        nc = 2
        wid = lax.axis_index("s") * nc + lax.axis_index("c")

        def chunk(c, _):
            base = wid * TPW + c * CHT
            pltpu.sync_copy(sp_hbm.at[pl.ds(base, CHT)], acc_v)
            pltpu.sync_copy(i0_hbm.at[wid, c], idx0_v)
            pltpu.sync_copy(i1_hbm.at[wid, c], idx1_v)
            cp0 = pltpu.async_copy(ys_hbm.at[idx0_v], r0_v, sem)
            cp1 = pltpu.async_copy(ys_hbm.at[idx1_v], r1_v, sem)
            cp0.wait()
            cp1.wait()

            def vstep(j, _):
                sl = pl.ds(j * 16, 16)
                for r in range(CHT):
                    acc_v[r, sl] = acc_v[r, sl] + (r0_v[r, sl] + r1_v[r, sl])
                return 0

            lax.fori_loop(0, D // 16, vstep, 0)
            pltpu.sync_copy(acc_v, out_hbm.at[pl.ds(base, CHT)])
            return 0

        lax.fori_loop(0, NCH, chunk, 0)

    return body(ys, shared_pre, i0, i1)


def kernel(hidden_states, router_weight, expert_gate_w, expert_up_w,
           expert_down_w, shared_gate_w, shared_up_w, shared_down_w,
           shared_gate_lin_w):
    b, s, d = hidden_states.shape
    x = hidden_states.reshape(T, D)

    dst, wts, meta, ws = _routing(x, router_weight)

    i0 = dst[:, 0].reshape(NW, NCH, CHT)
    i1 = dst[:, 1].reshape(NW, NCH, CHT)
    meta128 = meta.reshape(128)

    xs = _dispatch(x, i0, i1)
    shared_pre = _shared(x, shared_gate_w, shared_up_w, shared_down_w,
                         shared_gate_lin_w)
    ys = _grouped(meta128, xs, expert_gate_w, expert_up_w, expert_down_w,
                  ws.reshape(ROWS, 1))
    out = _combine(ys, shared_pre, i0, i1)
    return out.reshape(b, s, d)


# speedup vs baseline: 1.4526x; 1.0456x over previous
import functools

import jax
import jax.numpy as jnp
from jax import lax
from jax.experimental import pallas as pl
from jax.experimental.pallas import tpu as pltpu
from jax.experimental.pallas import tpu_sc as plsc

E = 8
TOPK = 2
D = 2048
FF = 512
FFS = 512
T = 2048
TILE_M = 128
NT = 40
ROWS = NT * TILE_M
NW = 32
TPW = T // NW
CHT = 16
NCH = TPW // CHT
CHC = 8
NCC = TPW // CHC


def _routing_body(x_ref, rw_ref, dst_ref, wts_ref, meta_ref, ws_ref, contrib_ref, cum_ref):
    x = x_ref[...]
    rw = rw_ref[...]
    logits = lax.dot_general(x, rw, (((1,), (1,)), ((), ())),
                             preferred_element_type=jnp.float32)
    m = jnp.max(logits, axis=1, keepdims=True)
    ex = jnp.exp(logits - m)
    probs = ex / jnp.sum(ex, axis=1, keepdims=True)

    iota8 = lax.broadcasted_iota(jnp.int32, (T, E), 1)
    v1 = jnp.max(probs, axis=1, keepdims=True)
    i1 = jnp.min(jnp.where(probs == v1, iota8, E), axis=1, keepdims=True)
    p2 = jnp.where(iota8 == i1, -1.0, probs)
    v2 = jnp.max(p2, axis=1, keepdims=True)
    i2 = jnp.min(jnp.where(p2 == v2, iota8, E), axis=1, keepdims=True)
    denom = v1 + v2
    w1 = v1 / denom
    w2 = v2 / denom

    oh1 = (iota8 == i1).astype(jnp.float32)
    oh2 = (iota8 == i2).astype(jnp.float32)
    contrib_ref[...] = oh1 + oh2

    CH = 256
    r = lax.broadcasted_iota(jnp.int32, (CH, CH), 0)
    c = lax.broadcasted_iota(jnp.int32, (CH, CH), 1)
    tri = (r > c).astype(jnp.float32)

    def step(ch, carry):
        blk = contrib_ref[pl.ds(ch * CH, CH), :]
        cum_blk = lax.dot_general(tri, blk, (((1,), (0,)), ((), ())),
                                  preferred_element_type=jnp.float32)
        cum_ref[pl.ds(ch * CH, CH), :] = cum_blk + carry
        return carry + jnp.sum(blk, axis=0, keepdims=True)

    counts = lax.fori_loop(0, T // CH, step, jnp.zeros((1, E), jnp.float32))

    ci = counts.astype(jnp.int32)
    ntiles = (ci + (TILE_M - 1)) // TILE_M
    e_r = lax.broadcasted_iota(jnp.int32, (E, E), 0)
    e_c = lax.broadcasted_iota(jnp.int32, (E, E), 1)
    mlt = (e_r < e_c).astype(jnp.float32)
    tile_off = lax.dot_general(ntiles.astype(jnp.float32), mlt,
                               (((1,), (0,)), ((), ())),
                               preferred_element_type=jnp.float32)
    row_off = tile_off * float(TILE_M)

    cum = cum_ref[...]
    sel1 = (iota8 == i1).astype(jnp.float32)
    sel2 = (iota8 == i2).astype(jnp.float32)
    dst0 = jnp.sum((cum + row_off) * sel1, axis=1, keepdims=True)
    dst1 = jnp.sum((cum + row_off) * sel2, axis=1, keepdims=True)

    colsel0 = (iota8 == 0).astype(jnp.float32)
    colsel1 = (iota8 == 1).astype(jnp.float32)
    dst_ref[...] = (dst0 * colsel0 + dst1 * colsel1).astype(jnp.int32)
    wts_ref[...] = w1 * colsel0 + w2 * colsel1

    WCH = 512
    piota = lax.broadcasted_iota(jnp.int32, (T, WCH), 1).astype(jnp.float32)
    for c in range(ROWS // WCH):
        pc = piota + float(c * WCH)
        contrib_w = (jnp.where(dst0 == pc, w1, 0.0)
                     + jnp.where(dst1 == pc, w2, 0.0))
        ws_ref[pl.ds(c, 1), :] = jnp.sum(contrib_w, axis=0, keepdims=True)

    tile_end = (tile_off + ntiles.astype(jnp.float32))
    i8 = (e_r == e_c).astype(jnp.float32)
    ends_col = lax.dot_general(i8, tile_end, (((1,), (1,)), ((), ())),
                               preferred_element_type=jnp.float32)
    ends_b = jnp.broadcast_to(ends_col, (E, 128))
    jot = lax.broadcasted_iota(jnp.int32, (E, 128), 1).astype(jnp.float32)
    eid = jnp.sum((ends_b <= jot).astype(jnp.float32), axis=0, keepdims=True)
    eid = jnp.minimum(eid, float(E - 1))
    n_active = jnp.sum(ntiles.astype(jnp.float32), axis=1, keepdims=True)
    lane = lax.broadcasted_iota(jnp.int32, (1, 128), 1)
    meta_ref[...] = jnp.where(lane == 120, n_active, eid).astype(jnp.int32)


def _routing(x, router_weight):
    return pl.pallas_call(
        _routing_body,
        out_shape=(
            jax.ShapeDtypeStruct((T, E), jnp.int32),
            jax.ShapeDtypeStruct((T, E), jnp.float32),
            jax.ShapeDtypeStruct((1, 128), jnp.int32),
            jax.ShapeDtypeStruct((ROWS // 512, 512), jnp.float32),
        ),
        scratch_shapes=[
            pltpu.VMEM((T, E), jnp.float32),
            pltpu.VMEM((T, E), jnp.float32),
        ],
    )(x, router_weight)


def _shared_body(x_ref, gw_ref, uw_ref, dw_ref, gl_ref, out_ref):
    x = x_ref[...]
    g = lax.dot_general(x, gw_ref[...], (((1,), (1,)), ((), ())),
                        preferred_element_type=jnp.float32)
    u = lax.dot_general(x, uw_ref[...], (((1,), (1,)), ((), ())),
                        preferred_element_type=jnp.float32)
    h = g * jax.nn.sigmoid(g) * u
    y = lax.dot_general(h, dw_ref[...], (((1,), (1,)), ((), ())),
                        preferred_element_type=jnp.float32)
    gate = jax.nn.sigmoid(lax.dot_general(x, gl_ref[...], (((1,), (1,)), ((), ())),
                                          preferred_element_type=jnp.float32))
    out_ref[...] = gate * y


def _shared(x, gw, uw, dw, gl):
    BT = 256
    return pl.pallas_call(
        _shared_body,
        grid=(T // BT,),
        in_specs=[
            pl.BlockSpec((BT, D), lambda i: (i, 0)),
            pl.BlockSpec((FFS, D), lambda i: (0, 0)),
            pl.BlockSpec((FFS, D), lambda i: (0, 0)),
            pl.BlockSpec((D, FFS), lambda i: (0, 0)),
            pl.BlockSpec((1, D), lambda i: (0, 0)),
        ],
        out_specs=pl.BlockSpec((BT, D), lambda i: (i, 0)),
        out_shape=jax.ShapeDtypeStruct((T, D), jnp.float32),
    )(x, gw, uw, dw, gl)


def _dispatch(x, i0, i1):
    mesh = plsc.VectorSubcoreMesh(core_axis_name="c", subcore_axis_name="s", num_cores=2, num_subcores=16)

    @functools.partial(
        pl.kernel,
        mesh=mesh,
        out_type=jax.ShapeDtypeStruct((ROWS, D), jnp.float32),
        scratch_types=[
            pltpu.VMEM((CHT, D), jnp.float32),
            pltpu.VMEM((CHT, D), jnp.float32),
            pltpu.VMEM((CHT,), jnp.int32),
            pltpu.VMEM((CHT,), jnp.int32),
            pltpu.VMEM((CHT,), jnp.int32),
            pltpu.VMEM((CHT,), jnp.int32),
            pltpu.SemaphoreType.DMA,
            pltpu.SemaphoreType.DMA,
            pltpu.SemaphoreType.DMA,
            pltpu.SemaphoreType.DMA,
        ],
    )
    def body(x_hbm, i0_hbm, i1_hbm, xs_hbm, rows_v0, rows_v1,
             idx0_v0, idx0_v1, idx1_v0, idx1_v1, semL0, semL1, semS0, semS1):
        nc = 2
        wid = lax.axis_index("s") * nc + lax.axis_index("c")
        rows = (rows_v0, rows_v1)
        idx0 = (idx0_v0, idx0_v1)
        idx1 = (idx1_v0, idx1_v1)
        semL = (semL0, semL1)
        semS = (semS0, semS1)

        def issue_load(c):
            b = c & 1
            base = wid * TPW + c * CHT
            return pltpu.async_copy(x_hbm.at[pl.ds(base, CHT)], rows[b], semL[b])

        ldp = issue_load(0)
        scp = None
        for c in range(NCH):
            b = c & 1
            ldp.wait()
            pltpu.sync_copy(i0_hbm.at[wid, c], idx0[b])
            pltpu.sync_copy(i1_hbm.at[wid, c], idx1[b])
            if scp is not None:
                scp[0].wait()
                scp[1].wait()
            if c + 1 < NCH:
                ldp = issue_load(c + 1)
            scp = (pltpu.async_copy(rows[b], xs_hbm.at[idx0[b]], semS[b]),
                   pltpu.async_copy(rows[b], xs_hbm.at[idx1[b]], semS[b]))
        scp[0].wait()
        scp[1].wait()

    return body(x, i0, i1)


def _grouped_body(sp_ref, xs_ref, gw_ref, uw_ref, dw_ref, ws_ref, out_ref):
    i = pl.program_id(0)
    n_active = sp_ref[120]

    @pl.when(i < n_active)
    def _():
        xb = xs_ref[...]
        gwe = gw_ref[0]
        uwe = uw_ref[0]
        dwe = dw_ref[0]
        g = lax.dot_general(xb, gwe, (((1,), (1,)), ((), ())),
                            preferred_element_type=jnp.float32)
        u = lax.dot_general(xb, uwe, (((1,), (1,)), ((), ())),
                            preferred_element_type=jnp.float32)
        h = g * jax.nn.sigmoid(g) * u
        h = h * ws_ref[...]
        out_ref[...] = lax.dot_general(h, dwe, (((1,), (1,)), ((), ())),
                                       preferred_element_type=jnp.float32)


def _grouped(meta128, xs, gw, uw, dw, ws2d):
    grid_spec = pltpu.PrefetchScalarGridSpec(
        num_scalar_prefetch=1,
        grid=(NT,),
        in_specs=[
            pl.BlockSpec((TILE_M, D), lambda i, sp: (i, 0)),
            pl.BlockSpec((1, FF, D), lambda i, sp: (sp[i], 0, 0)),
            pl.BlockSpec((1, FF, D), lambda i, sp: (sp[i], 0, 0)),
            pl.BlockSpec((1, D, FF), lambda i, sp: (sp[i], 0, 0)),
            pl.BlockSpec((TILE_M, 1), lambda i, sp: (i, 0)),
        ],
        out_specs=pl.BlockSpec((TILE_M, D), lambda i, sp: (i, 0)),
    )
    return pl.pallas_call(
        _grouped_body,
        grid_spec=grid_spec,
        out_shape=jax.ShapeDtypeStruct((ROWS, D), jnp.float32),
    )(meta128, xs, gw, uw, dw, ws2d)


def _combine(ys, shared_pre, i0, i1):
    mesh = plsc.VectorSubcoreMesh(core_axis_name="c", subcore_axis_name="s", num_cores=2, num_subcores=16)

    @functools.partial(
        pl.kernel,
        mesh=mesh,
        out_type=jax.ShapeDtypeStruct((T, D), jnp.float32),
        scratch_types=[
            pltpu.VMEM((CHC, D), jnp.float32),
            pltpu.VMEM((CHC, D), jnp.float32),
            pltpu.VMEM((CHC, D), jnp.float32),
            pltpu.VMEM((CHC, D), jnp.float32),
            pltpu.VMEM((CHC, D), jnp.float32),
            pltpu.VMEM((CHC, D), jnp.float32),
            pltpu.VMEM((CHC,), jnp.int32),
            pltpu.VMEM((CHC,), jnp.int32),
            pltpu.VMEM((CHC,), jnp.int32),
            pltpu.VMEM((CHC,), jnp.int32),
            pltpu.SemaphoreType.DMA,
            pltpu.SemaphoreType.DMA,
        ],
    )
    def body(ys_hbm, sp_hbm, i0_hbm, i1_hbm, out_hbm,
             acc_v0, acc_v1, r0_v0, r0_v1, r1_v0, r1_v1,
             idx0_v0, idx0_v1, idx1_v0, idx1_v1, sem0, sem1):
        nc = 2
        wid = lax.axis_index("s") * nc + lax.axis_index("c")
        acc = (acc_v0, acc_v1)
        r0 = (r0_v0, r0_v1)
        r1 = (r1_v0, r1_v1)
        idx0 = (idx0_v0, idx0_v1)
        idx1 = (idx1_v0, idx1_v1)
        sems = (sem0, sem1)

        def issue(c):
            b = c & 1
            base = wid * TPW + c * CHC
            pltpu.sync_copy(i0_hbm.at[wid, c], idx0[b])
            pltpu.sync_copy(i1_hbm.at[wid, c], idx1[b])
            return (pltpu.async_copy(sp_hbm.at[pl.ds(base, CHC)], acc[b], sems[b]),
                    pltpu.async_copy(ys_hbm.at[idx0[b]], r0[b], sems[b]),
                    pltpu.async_copy(ys_hbm.at[idx1[b]], r1[b], sems[b]))

        pend = issue(0)
        for c in range(NCC):
            b = c & 1
            nxt = issue(c + 1) if c + 1 < NCC else None
            for cp in pend:
                cp.wait()

            def vstep(j, _):
                sl = pl.ds(j * 16, 16)
                for r in range(CHC):
                    acc[b][r, sl] = acc[b][r, sl] + (r0[b][r, sl] + r1[b][r, sl])
                return 0

            lax.fori_loop(0, D // 16, vstep, 0)
            base = wid * TPW + c * CHC
            pltpu.sync_copy(acc[b], out_hbm.at[pl.ds(base, CHC)])
            pend = nxt

    return body(ys, shared_pre, i0, i1)


def kernel(hidden_states, router_weight, expert_gate_w, expert_up_w,
           expert_down_w, shared_gate_w, shared_up_w, shared_down_w,
           shared_gate_lin_w):
    b, s, d = hidden_states.shape
    x = hidden_states.reshape(T, D)

    dst, wts, meta, ws = _routing(x, router_weight)

    i0 = dst[:, 0].reshape(NW, NCH, CHT)
    i1 = dst[:, 1].reshape(NW, NCH, CHT)
    i0c = dst[:, 0].reshape(NW, NCC, CHC)
    i1c = dst[:, 1].reshape(NW, NCC, CHC)
    meta128 = meta.reshape(128)

    xs = _dispatch(x, i0, i1)
    shared_pre = _shared(x, shared_gate_w, shared_up_w, shared_down_w,
                         shared_gate_lin_w)
    ys = _grouped(meta128, xs, expert_gate_w, expert_up_w, expert_down_w,
                  ws.reshape(ROWS, 1))
    out = _combine(ys, shared_pre, i0c, i1c)
    return out.reshape(b, s, d)


# speedup vs baseline: 1.6193x; 1.1148x over previous
import functools

import jax
import jax.numpy as jnp
from jax import lax
from jax.experimental import pallas as pl
from jax.experimental.pallas import tpu as pltpu
from jax.experimental.pallas import tpu_sc as plsc

E = 8
TOPK = 2
D = 2048
FF = 512
FFS = 512
T = 2048
TILE_M = 512
NT = 16
ROWS = NT * TILE_M
NW = 32
TPW = T // NW
CHT = 16
NCH = TPW // CHT
CHC = 8
NCC = TPW // CHC


def _routing_body(x_ref, rw_ref, dst_ref, wts_ref, meta_ref, ws_ref, contrib_ref, cum_ref):
    x = x_ref[...]
    rw = rw_ref[...]
    logits = lax.dot_general(x, rw, (((1,), (1,)), ((), ())),
                             preferred_element_type=jnp.float32)
    m = jnp.max(logits, axis=1, keepdims=True)
    ex = jnp.exp(logits - m)
    probs = ex / jnp.sum(ex, axis=1, keepdims=True)

    iota8 = lax.broadcasted_iota(jnp.int32, (T, E), 1)
    v1 = jnp.max(probs, axis=1, keepdims=True)
    i1 = jnp.min(jnp.where(probs == v1, iota8, E), axis=1, keepdims=True)
    p2 = jnp.where(iota8 == i1, -1.0, probs)
    v2 = jnp.max(p2, axis=1, keepdims=True)
    i2 = jnp.min(jnp.where(p2 == v2, iota8, E), axis=1, keepdims=True)
    denom = v1 + v2
    w1 = v1 / denom
    w2 = v2 / denom

    oh1 = (iota8 == i1).astype(jnp.float32)
    oh2 = (iota8 == i2).astype(jnp.float32)
    contrib_ref[...] = oh1 + oh2

    CH = 256
    r = lax.broadcasted_iota(jnp.int32, (CH, CH), 0)
    c = lax.broadcasted_iota(jnp.int32, (CH, CH), 1)
    tri = (r > c).astype(jnp.float32)

    def step(ch, carry):
        blk = contrib_ref[pl.ds(ch * CH, CH), :]
        cum_blk = lax.dot_general(tri, blk, (((1,), (0,)), ((), ())),
                                  preferred_element_type=jnp.float32)
        cum_ref[pl.ds(ch * CH, CH), :] = cum_blk + carry
        return carry + jnp.sum(blk, axis=0, keepdims=True)

    counts = lax.fori_loop(0, T // CH, step, jnp.zeros((1, E), jnp.float32))

    ci = counts.astype(jnp.int32)
    ntiles = (ci + (TILE_M - 1)) // TILE_M
    e_r = lax.broadcasted_iota(jnp.int32, (E, E), 0)
    e_c = lax.broadcasted_iota(jnp.int32, (E, E), 1)
    mlt = (e_r < e_c).astype(jnp.float32)
    tile_off = lax.dot_general(ntiles.astype(jnp.float32), mlt,
                               (((1,), (0,)), ((), ())),
                               preferred_element_type=jnp.float32)
    row_off = tile_off * float(TILE_M)

    cum = cum_ref[...]
    sel1 = (iota8 == i1).astype(jnp.float32)
    sel2 = (iota8 == i2).astype(jnp.float32)
    dst0 = jnp.sum((cum + row_off) * sel1, axis=1, keepdims=True)
    dst1 = jnp.sum((cum + row_off) * sel2, axis=1, keepdims=True)

    colsel0 = (iota8 == 0).astype(jnp.float32)
    colsel1 = (iota8 == 1).astype(jnp.float32)
    dst_ref[...] = (dst0 * colsel0 + dst1 * colsel1).astype(jnp.int32)
    wts_ref[...] = w1 * colsel0 + w2 * colsel1

    WCH = 512
    piota = lax.broadcasted_iota(jnp.int32, (T, WCH), 1).astype(jnp.float32)
    for c in range(ROWS // WCH):
        pc = piota + float(c * WCH)
        contrib_w = (jnp.where(dst0 == pc, w1, 0.0)
                     + jnp.where(dst1 == pc, w2, 0.0))
        ws_ref[pl.ds(c, 1), :] = jnp.sum(contrib_w, axis=0, keepdims=True)

    tile_end = (tile_off + ntiles.astype(jnp.float32))
    i8 = (e_r == e_c).astype(jnp.float32)
    ends_col = lax.dot_general(i8, tile_end, (((1,), (1,)), ((), ())),
                               preferred_element_type=jnp.float32)
    ends_b = jnp.broadcast_to(ends_col, (E, 128))
    jot = lax.broadcasted_iota(jnp.int32, (E, 128), 1).astype(jnp.float32)
    eid = jnp.sum((ends_b <= jot).astype(jnp.float32), axis=0, keepdims=True)
    eid = jnp.minimum(eid, float(E - 1))
    n_active = jnp.sum(ntiles.astype(jnp.float32), axis=1, keepdims=True)
    lane = lax.broadcasted_iota(jnp.int32, (1, 128), 1)
    meta_ref[...] = jnp.where(lane == 120, n_active, eid).astype(jnp.int32)


def _routing(x, router_weight):
    return pl.pallas_call(
        _routing_body,
        out_shape=(
            jax.ShapeDtypeStruct((T, E), jnp.int32),
            jax.ShapeDtypeStruct((T, E), jnp.float32),
            jax.ShapeDtypeStruct((1, 128), jnp.int32),
            jax.ShapeDtypeStruct((ROWS // 512, 512), jnp.float32),
        ),
        scratch_shapes=[
            pltpu.VMEM((T, E), jnp.float32),
            pltpu.VMEM((T, E), jnp.float32),
        ],
    )(x, router_weight)


def _shared_body(x_ref, gw_ref, uw_ref, dw_ref, gl_ref, out_ref):
    x = x_ref[...]
    g = lax.dot_general(x, gw_ref[...], (((1,), (1,)), ((), ())),
                        preferred_element_type=jnp.float32)
    u = lax.dot_general(x, uw_ref[...], (((1,), (1,)), ((), ())),
                        preferred_element_type=jnp.float32)
    h = g * jax.nn.sigmoid(g) * u
    y = lax.dot_general(h, dw_ref[...], (((1,), (1,)), ((), ())),
                        preferred_element_type=jnp.float32)
    gate = jax.nn.sigmoid(lax.dot_general(x, gl_ref[...], (((1,), (1,)), ((), ())),
                                          preferred_element_type=jnp.float32))
    out_ref[...] = gate * y


def _shared(x, gw, uw, dw, gl):
    BT = 1024
    return pl.pallas_call(
        _shared_body,
        grid=(T // BT,),
        in_specs=[
            pl.BlockSpec((BT, D), lambda i: (i, 0)),
            pl.BlockSpec((FFS, D), lambda i: (0, 0)),
            pl.BlockSpec((FFS, D), lambda i: (0, 0)),
            pl.BlockSpec((D, FFS), lambda i: (0, 0)),
            pl.BlockSpec((1, D), lambda i: (0, 0)),
        ],
        out_specs=pl.BlockSpec((BT, D), lambda i: (i, 0)),
        out_shape=jax.ShapeDtypeStruct((T, D), jnp.float32),
    )(x, gw, uw, dw, gl)


def _dispatch(x, i0, i1):
    mesh = plsc.VectorSubcoreMesh(core_axis_name="c", subcore_axis_name="s", num_cores=2, num_subcores=16)

    @functools.partial(
        pl.kernel,
        mesh=mesh,
        out_type=jax.ShapeDtypeStruct((ROWS, D), jnp.float32),
        scratch_types=[
            pltpu.VMEM((CHT, D), jnp.float32),
            pltpu.VMEM((CHT, D), jnp.float32),
            pltpu.VMEM((CHT,), jnp.int32),
            pltpu.VMEM((CHT,), jnp.int32),
            pltpu.VMEM((CHT,), jnp.int32),
            pltpu.VMEM((CHT,), jnp.int32),
            pltpu.SemaphoreType.DMA,
            pltpu.SemaphoreType.DMA,
            pltpu.SemaphoreType.DMA,
            pltpu.SemaphoreType.DMA,
        ],
    )
    def body(x_hbm, i0_hbm, i1_hbm, xs_hbm, rows_v0, rows_v1,
             idx0_v0, idx0_v1, idx1_v0, idx1_v1, semL0, semL1, semS0, semS1):
        nc = 2
        wid = lax.axis_index("s") * nc + lax.axis_index("c")
        rows = (rows_v0, rows_v1)
        idx0 = (idx0_v0, idx0_v1)
        idx1 = (idx1_v0, idx1_v1)
        semL = (semL0, semL1)
        semS = (semS0, semS1)

        def issue_load(c):
            b = c & 1
            base = wid * TPW + c * CHT
            return pltpu.async_copy(x_hbm.at[pl.ds(base, CHT)], rows[b], semL[b])

        ldp = issue_load(0)
        scp = None
        for c in range(NCH):
            b = c & 1
            ldp.wait()
            pltpu.sync_copy(i0_hbm.at[wid, c], idx0[b])
            pltpu.sync_copy(i1_hbm.at[wid, c], idx1[b])
            if scp is not None:
                scp[0].wait()
                scp[1].wait()
            if c + 1 < NCH:
                ldp = issue_load(c + 1)
            scp = (pltpu.async_copy(rows[b], xs_hbm.at[idx0[b]], semS[b]),
                   pltpu.async_copy(rows[b], xs_hbm.at[idx1[b]], semS[b]))
        scp[0].wait()
        scp[1].wait()

    return body(x, i0, i1)


def _grouped_body(sp_ref, xs_ref, gw_ref, uw_ref, dw_ref, ws_ref, out_ref):
    i = pl.program_id(0)
    n_active = sp_ref[120]

    @pl.when(i < n_active)
    def _():
        xb = xs_ref[...]
        gwe = gw_ref[0]
        uwe = uw_ref[0]
        dwe = dw_ref[0]
        g = lax.dot_general(xb, gwe, (((1,), (1,)), ((), ())),
                            preferred_element_type=jnp.float32)
        u = lax.dot_general(xb, uwe, (((1,), (1,)), ((), ())),
                            preferred_element_type=jnp.float32)
        h = g * jax.nn.sigmoid(g) * u
        h = h * ws_ref[...]
        out_ref[...] = lax.dot_general(h, dwe, (((1,), (1,)), ((), ())),
                                       preferred_element_type=jnp.float32)


def _grouped(meta128, xs, gw, uw, dw, ws2d):
    grid_spec = pltpu.PrefetchScalarGridSpec(
        num_scalar_prefetch=1,
        grid=(NT,),
        in_specs=[
            pl.BlockSpec((TILE_M, D), lambda i, sp: (i, 0)),
            pl.BlockSpec((1, FF, D), lambda i, sp: (sp[i], 0, 0)),
            pl.BlockSpec((1, FF, D), lambda i, sp: (sp[i], 0, 0)),
            pl.BlockSpec((1, D, FF), lambda i, sp: (sp[i], 0, 0)),
            pl.BlockSpec((TILE_M, 1), lambda i, sp: (i, 0)),
        ],
        out_specs=pl.BlockSpec((TILE_M, D), lambda i, sp: (i, 0)),
    )
    return pl.pallas_call(
        _grouped_body,
        grid_spec=grid_spec,
        out_shape=jax.ShapeDtypeStruct((ROWS, D), jnp.float32),
    )(meta128, xs, gw, uw, dw, ws2d)


def _combine(ys, shared_pre, i0, i1):
    mesh = plsc.VectorSubcoreMesh(core_axis_name="c", subcore_axis_name="s", num_cores=2, num_subcores=16)

    @functools.partial(
        pl.kernel,
        mesh=mesh,
        out_type=jax.ShapeDtypeStruct((T, D), jnp.float32),
        scratch_types=[
            pltpu.VMEM((CHC, D), jnp.float32),
            pltpu.VMEM((CHC, D), jnp.float32),
            pltpu.VMEM((CHC, D), jnp.float32),
            pltpu.VMEM((CHC, D), jnp.float32),
            pltpu.VMEM((CHC, D), jnp.float32),
            pltpu.VMEM((CHC, D), jnp.float32),
            pltpu.VMEM((CHC,), jnp.int32),
            pltpu.VMEM((CHC,), jnp.int32),
            pltpu.VMEM((CHC,), jnp.int32),
            pltpu.VMEM((CHC,), jnp.int32),
            pltpu.SemaphoreType.DMA,
            pltpu.SemaphoreType.DMA,
        ],
    )
    def body(ys_hbm, sp_hbm, i0_hbm, i1_hbm, out_hbm,
             acc_v0, acc_v1, r0_v0, r0_v1, r1_v0, r1_v1,
             idx0_v0, idx0_v1, idx1_v0, idx1_v1, sem0, sem1):
        nc = 2
        wid = lax.axis_index("s") * nc + lax.axis_index("c")
        acc = (acc_v0, acc_v1)
        r0 = (r0_v0, r0_v1)
        r1 = (r1_v0, r1_v1)
        idx0 = (idx0_v0, idx0_v1)
        idx1 = (idx1_v0, idx1_v1)
        sems = (sem0, sem1)

        def issue(c):
            b = c & 1
            base = wid * TPW + c * CHC
            pltpu.sync_copy(i0_hbm.at[wid, c], idx0[b])
            pltpu.sync_copy(i1_hbm.at[wid, c], idx1[b])
            return (pltpu.async_copy(sp_hbm.at[pl.ds(base, CHC)], acc[b], sems[b]),
                    pltpu.async_copy(ys_hbm.at[idx0[b]], r0[b], sems[b]),
                    pltpu.async_copy(ys_hbm.at[idx1[b]], r1[b], sems[b]))

        pend = issue(0)
        for c in range(NCC):
            b = c & 1
            nxt = issue(c + 1) if c + 1 < NCC else None
            for cp in pend:
                cp.wait()

            def vstep(j, _):
                sl = pl.ds(j * 16, 16)
                for r in range(CHC):
                    acc[b][r, sl] = acc[b][r, sl] + (r0[b][r, sl] + r1[b][r, sl])
                return 0

            lax.fori_loop(0, D // 16, vstep, 0)
            base = wid * TPW + c * CHC
            pltpu.sync_copy(acc[b], out_hbm.at[pl.ds(base, CHC)])
            pend = nxt

    return body(ys, shared_pre, i0, i1)


def kernel(hidden_states, router_weight, expert_gate_w, expert_up_w,
           expert_down_w, shared_gate_w, shared_up_w, shared_down_w,
           shared_gate_lin_w):
    b, s, d = hidden_states.shape
    x = hidden_states.reshape(T, D)

    dst, wts, meta, ws = _routing(x, router_weight)

    i0 = dst[:, 0].reshape(NW, NCH, CHT)
    i1 = dst[:, 1].reshape(NW, NCH, CHT)
    i0c = dst[:, 0].reshape(NW, NCC, CHC)
    i1c = dst[:, 1].reshape(NW, NCC, CHC)
    meta128 = meta.reshape(128)

    xs = _dispatch(x, i0, i1)
    shared_pre = _shared(x, shared_gate_w, shared_up_w, shared_down_w,
                         shared_gate_lin_w)
    ys = _grouped(meta128, xs, expert_gate_w, expert_up_w, expert_down_w,
                  ws.reshape(ROWS, 1))
    out = _combine(ys, shared_pre, i0c, i1c)
    return out.reshape(b, s, d)
